# baseline (speedup 1.0000x reference)
import functools

import jax
import jax.numpy as jnp
from jax import lax
from jax.experimental import pallas as pl
from jax.experimental.pallas import tpu as pltpu
from jax.experimental.pallas import tpu_sc as plsc

_CHUNK = 128


def _tc_project(x, bases, coeff, W_self, bias2d):
    N, D = x.shape
    R, B = coeff.shape
    NB = 400
    assert N % NB == 0

    def body(coeff_ref, x_ref, bases_ref, wself_ref, bias_ref, p_ref, s_ref):
        xb = x_ref[...]
        projs = [jnp.dot(xb, bases_ref[b], preferred_element_type=jnp.float32)
                 for b in range(B)]
        for r in range(R):
            acc = projs[0] * coeff_ref[r, 0]
            for b in range(1, B):
                acc = acc + projs[b] * coeff_ref[r, b]
            p_ref[:, r * D:(r + 1) * D] = acc
        s_ref[...] = (jnp.dot(xb, wself_ref[...],
                              preferred_element_type=jnp.float32)
                      + bias_ref[...])

    return pl.pallas_call(
        body,
        grid=(N // NB,),
        in_specs=[
            pl.BlockSpec(memory_space=pltpu.SMEM),
            pl.BlockSpec((NB, D), lambda i: (i, 0)),
            pl.BlockSpec((B, D, D), lambda i: (0, 0, 0)),
            pl.BlockSpec((D, D), lambda i: (0, 0)),
            pl.BlockSpec((1, D), lambda i: (0, 0)),
        ],
        out_specs=[
            pl.BlockSpec((NB, R * D), lambda i: (i, 0)),
            pl.BlockSpec((NB, D), lambda i: (i, 0)),
        ],
        out_shape=[
            jax.ShapeDtypeStruct((N, R * D), jnp.float32),
            jax.ShapeDtypeStruct((N, D), jnp.float32),
        ],
    )(coeff, x, bases, W_self, bias2d)


def _sc_edge_aggregate(p_flat, src2, et2, dst2, R, NH, CH):
    info = plsc.get_sparse_core_info()
    NC, NS = info.num_cores, info.num_subcores
    D = p_flat.shape[1]
    rows_per_tile = NH // NS
    n_drain = rows_per_tile // _CHUNK
    assert CH % 2 == 0
    mesh = plsc.VectorSubcoreMesh(core_axis_name="c", subcore_axis_name="s")

    @functools.partial(
        pl.kernel,
        out_type=jax.ShapeDtypeStruct((NC * NH, D), jnp.float32),
        mesh=mesh,
        scratch_types=[
            [pltpu.VMEM((_CHUNK,), jnp.int32)] * 2,
            [pltpu.VMEM((_CHUNK,), jnp.int32)] * 2,
            [pltpu.VMEM((_CHUNK,), jnp.int32)] * 2,
            [pltpu.VMEM((_CHUNK,), jnp.int32)] * 2,
            [pltpu.VMEM((_CHUNK, D), jnp.float32)] * 2,
            pltpu.VMEM_SHARED((NH, D), jnp.float32),
            [pltpu.SemaphoreType.DMA] * 2,
            [pltpu.SemaphoreType.DMA] * 2,
        ],
    )
    def k(p_hbm, src_hbm, et_hbm, dst_hbm, out_hbm,
          srcb, etb, dstb, gidxb, rows, h_sh, msem, gsem):
        cid = lax.axis_index("c")
        sid = lax.axis_index("s")
        wid = sid * NC + cid
        base_row = wid * CH

        def mfire(j, s):
            pltpu.async_copy(src_hbm.at[base_row + j], srcb[s], msem[s])
            pltpu.async_copy(et_hbm.at[base_row + j], etb[s], msem[s])
            pltpu.async_copy(dst_hbm.at[base_row + j], dstb[s], msem[s])

        def mwait(s):
            for buf in (srcb[s], etb[s], dstb[s]):
                pltpu.make_async_copy(src_hbm.at[0], buf, msem[s]).wait()

        def gidxc(s):
            for i in range(_CHUNK // 16):
                sl = pl.ds(i * 16, 16)
                gidxb[s][sl] = srcb[s][sl] * R + etb[s][sl]

        def gfire(s):
            pltpu.async_copy(p_hbm.at[gidxb[s]], rows[s], gsem[s])

        def gwait(s):
            pltpu.make_async_copy(p_hbm.at[pl.ds(0, _CHUNK)], rows[s],
                                  gsem[s]).wait()

        def scatter(s):
            pltpu.sync_copy(rows[s], h_sh.at[dstb[s]], add=True)

        zero16 = jnp.zeros((16,), jnp.float32)
        nlane = D // 16

        def zrow(i, _):
            rows[0][i // nlane, pl.ds((i % nlane) * 16, 16)] = zero16
            return 0
        lax.fori_loop(0, _CHUNK * nlane, zrow, 0)

        stripe = sid * rows_per_tile

        def zcopy(t, _):
            pltpu.sync_copy(rows[0], h_sh.at[pl.ds(stripe + t * _CHUNK,
                                                   _CHUNK)])
            return 0
        lax.fori_loop(0, n_drain, zcopy, 0)
        plsc.subcore_barrier()

        T = CH // 2
        mfire(0, 0)
        mwait(0)
        gidxc(0)
        gfire(0)
        mfire(1, 1)

        def pair(t, _):
            j = t * 2
            mwait(1)
            gidxc(1)
            gfire(1)
            gwait(0)
            scatter(0)

            @pl.when(t < T - 1)
            def _():
                mfire(j + 2, 0)
                mwait(0)
                gidxc(0)
                gfire(0)
            gwait(1)
            scatter(1)

            @pl.when(t < T - 1)
            def _():
                mfire(j + 3, 1)
            return 0
        lax.fori_loop(0, T, pair, 0)
        plsc.subcore_barrier()

        def dcopy(t, _):
            base = stripe + t * _CHUNK
            pltpu.sync_copy(h_sh.at[pl.ds(base, _CHUNK)], rows[0])
            pltpu.sync_copy(rows[0], out_hbm.at[pl.ds(cid * NH + base,
                                                      _CHUNK)])
            return 0
        lax.fori_loop(0, n_drain, dcopy, 0)

    return k(p_flat, src2, et2, dst2)


def _tc_combine(a, b, s):
    N, D = a.shape
    NB = 400

    def body(a_ref, b_ref, s_ref, o_ref):
        o_ref[...] = a_ref[...] + b_ref[...] + s_ref[...]

    spec = pl.BlockSpec((NB, D), lambda i: (i, 0))
    return pl.pallas_call(
        body,
        grid=(N // NB,),
        in_specs=[spec, spec, spec],
        out_specs=spec,
        out_shape=jax.ShapeDtypeStruct((N, D), jnp.float32),
    )(a, b, s)


def kernel(x, edge_index, etypes, bases, coeff, W_self, bias):
    N, D = x.shape
    E = edge_index.shape[1]
    R, B = coeff.shape

    info = plsc.get_sparse_core_info()
    NC, NS = info.num_cores, info.num_subcores
    NW = NC * NS

    NH = ((N + 1 + NS * _CHUNK - 1) // (NS * _CHUNK)) * (NS * _CHUNK)
    CH = (E + NW * _CHUNK - 1) // (NW * _CHUNK)
    CH = CH + (CH % 2)
    E_pad = NW * CH * _CHUNK

    p, self_term = _tc_project(x, bases, coeff, W_self, bias.reshape(1, D))
    p_flat = p.reshape(N * R, D)

    src = edge_index[0]
    dst = edge_index[1]
    pad = E_pad - E
    src2 = jnp.concatenate([src, jnp.zeros((pad,), jnp.int32)]).reshape(-1, _CHUNK)
    et2 = jnp.concatenate([etypes, jnp.zeros((pad,), jnp.int32)]).reshape(-1, _CHUNK)
    dst2 = jnp.concatenate([dst, jnp.full((pad,), N, jnp.int32)]).reshape(-1, _CHUNK)

    partial = _sc_edge_aggregate(p_flat, src2, et2, dst2, R, NH, CH)
    partial = partial.reshape(NC, NH, D)

    return _tc_combine(partial[0, :N], partial[1, :N], self_term)


# speedup vs baseline: 1.0001x; 1.0001x over previous
import functools

import jax
import jax.numpy as jnp
from jax import lax
from jax.experimental import pallas as pl
from jax.experimental.pallas import tpu as pltpu
from jax.experimental.pallas import tpu_sc as plsc

_CHUNK = 128


def _tc_project(x, bases, coeff, W_self, bias2d):
    N, D = x.shape
    R, B = coeff.shape
    NB = 400
    assert N % NB == 0

    def body(coeff_ref, x_ref, bases_ref, wself_ref, bias_ref, p_ref, s_ref):
        xb = x_ref[...]
        projs = [jnp.dot(xb, bases_ref[b], preferred_element_type=jnp.float32)
                 for b in range(B)]
        for r in range(R):
            acc = projs[0] * coeff_ref[r, 0]
            for b in range(1, B):
                acc = acc + projs[b] * coeff_ref[r, b]
            p_ref[:, r * D:(r + 1) * D] = acc
        s_ref[...] = (jnp.dot(xb, wself_ref[...],
                              preferred_element_type=jnp.float32)
                      + bias_ref[...])

    return pl.pallas_call(
        body,
        grid=(N // NB,),
        in_specs=[
            pl.BlockSpec(memory_space=pltpu.SMEM),
            pl.BlockSpec((NB, D), lambda i: (i, 0)),
            pl.BlockSpec((B, D, D), lambda i: (0, 0, 0)),
            pl.BlockSpec((D, D), lambda i: (0, 0)),
            pl.BlockSpec((1, D), lambda i: (0, 0)),
        ],
        out_specs=[
            pl.BlockSpec((NB, R * D), lambda i: (i, 0)),
            pl.BlockSpec((NB, D), lambda i: (i, 0)),
        ],
        out_shape=[
            jax.ShapeDtypeStruct((N, R * D), jnp.float32),
            jax.ShapeDtypeStruct((N, D), jnp.float32),
        ],
    )(coeff, x, bases, W_self, bias2d)


def _sc_edge_aggregate(p_flat, src2, et2, dst2, R, NH, CH):
    info = plsc.get_sparse_core_info()
    NC, NS = info.num_cores, info.num_subcores
    D = p_flat.shape[1]
    rows_per_tile = NH // NS
    n_drain = rows_per_tile // _CHUNK
    assert CH % 2 == 0
    mesh = plsc.VectorSubcoreMesh(core_axis_name="c", subcore_axis_name="s")

    MDEPTH = 8

    @functools.partial(
        pl.kernel,
        out_type=jax.ShapeDtypeStruct((NC * NH, D), jnp.float32),
        mesh=mesh,
        scratch_types=[
            [pltpu.VMEM((_CHUNK,), jnp.int32)] * MDEPTH,
            [pltpu.VMEM((_CHUNK,), jnp.int32)] * MDEPTH,
            [pltpu.VMEM((_CHUNK,), jnp.int32)] * MDEPTH,
            [pltpu.VMEM((_CHUNK,), jnp.int32)] * 2,
            [pltpu.VMEM((_CHUNK, D), jnp.float32)] * 2,
            pltpu.VMEM_SHARED((NH, D), jnp.float32),
            [pltpu.SemaphoreType.DMA] * MDEPTH,
            [pltpu.SemaphoreType.DMA] * 2,
            [pltpu.SemaphoreType.DMA] * 2,
        ],
    )
    def k(p_hbm, src_hbm, et_hbm, dst_hbm, out_hbm,
          srcb, etb, dstb, gidxb, rows, h_sh, msem, gsem, ssem):
        cid = lax.axis_index("c")
        sid = lax.axis_index("s")
        wid = sid * NC + cid
        base_row = wid * CH

        def mfire(j, s):
            pltpu.async_copy(src_hbm.at[base_row + j], srcb[s], msem[s])
            pltpu.async_copy(et_hbm.at[base_row + j], etb[s], msem[s])
            pltpu.async_copy(dst_hbm.at[base_row + j], dstb[s], msem[s])

        def mwait(s):
            for buf in (srcb[s], etb[s], dstb[s]):
                pltpu.make_async_copy(src_hbm.at[0], buf, msem[s]).wait()

        def gidxc(g, s):
            for i in range(_CHUNK // 16):
                sl = pl.ds(i * 16, 16)
                gidxb[g][sl] = srcb[s][sl] * R + etb[s][sl]

        def gfire(g):
            pltpu.async_copy(p_hbm.at[gidxb[g]], rows[g], gsem[g])

        def gwait(g):
            pltpu.make_async_copy(p_hbm.at[pl.ds(0, _CHUNK)], rows[g],
                                  gsem[g]).wait()

        def sfire(g, s):
            pltpu.async_copy(rows[g], h_sh.at[dstb[s]], ssem[g], add=True)

        def swait(g):
            pltpu.make_async_copy(p_hbm.at[pl.ds(0, _CHUNK)], rows[g],
                                  ssem[g]).wait()

        zero16 = jnp.zeros((16,), jnp.float32)
        nlane = D // 16

        def zrow(i, _):
            rows[0][i // nlane, pl.ds((i % nlane) * 16, 16)] = zero16
            return 0
        lax.fori_loop(0, _CHUNK * nlane, zrow, 0)

        stripe = sid * rows_per_tile

        def zcopy(t, _):
            pltpu.sync_copy(rows[0], h_sh.at[pl.ds(stripe + t * _CHUNK,
                                                   _CHUNK)])
            return 0
        lax.fori_loop(0, n_drain, zcopy, 0)
        plsc.subcore_barrier()

        T = CH // MDEPTH
        assert CH % MDEPTH == 0 and T >= 2

        def chunk_step(c, q, first_octet, last_octet):
            g, m = q % 2, q
            gn, mn = (q + 1) % 2, (q + 1) % MDEPTH
            mp = (q - 1) % MDEPTH
            has_next = not (last_octet and q == MDEPTH - 1)
            if has_next:
                mwait(mn)
                gidxc(gn, mn)
            if not (first_octet and q == 0):
                swait(gn)
                if (not last_octet) or q == 0:
                    mfire(c - 1 + MDEPTH, mp)
            if has_next:
                gfire(gn)
            gwait(g)
            sfire(g, m)

        for s in range(MDEPTH):
            mfire(s, s)
        mwait(0)
        gidxc(0, 0)
        gfire(0)

        for q in range(MDEPTH):
            chunk_step(q, q, True, T == 1)
        if T > 2:
            def octet(u, _):
                for q in range(MDEPTH):
                    chunk_step(u * MDEPTH + q, q, False, False)
                return 0
            lax.fori_loop(1, T - 1, octet, 0)
        if T > 1:
            for q in range(MDEPTH):
                chunk_step((T - 1) * MDEPTH + q, q, False, True)
        swait((CH - 1) % 2)
        plsc.subcore_barrier()

        def dcopy(t, _):
            base = stripe + t * _CHUNK
            pltpu.sync_copy(h_sh.at[pl.ds(base, _CHUNK)], rows[0])
            pltpu.sync_copy(rows[0], out_hbm.at[pl.ds(cid * NH + base,
                                                      _CHUNK)])
            return 0
        lax.fori_loop(0, n_drain, dcopy, 0)

    return k(p_flat, src2, et2, dst2)


def _tc_combine(a, b, s):
    N, D = a.shape
    NB = 400

    def body(a_ref, b_ref, s_ref, o_ref):
        o_ref[...] = a_ref[...] + b_ref[...] + s_ref[...]

    spec = pl.BlockSpec((NB, D), lambda i: (i, 0))
    return pl.pallas_call(
        body,
        grid=(N // NB,),
        in_specs=[spec, spec, spec],
        out_specs=spec,
        out_shape=jax.ShapeDtypeStruct((N, D), jnp.float32),
    )(a, b, s)


def kernel(x, edge_index, etypes, bases, coeff, W_self, bias):
    N, D = x.shape
    E = edge_index.shape[1]
    R, B = coeff.shape

    info = plsc.get_sparse_core_info()
    NC, NS = info.num_cores, info.num_subcores
    NW = NC * NS

    NH = ((N + 1 + NS * _CHUNK - 1) // (NS * _CHUNK)) * (NS * _CHUNK)
    CH = (E + NW * _CHUNK - 1) // (NW * _CHUNK)
    CH = max(((CH + 7) // 8) * 8, 16)
    E_pad = NW * CH * _CHUNK

    p, self_term = _tc_project(x, bases, coeff, W_self, bias.reshape(1, D))
    p_flat = p.reshape(N * R, D)

    src = edge_index[0]
    dst = edge_index[1]
    pad = E_pad - E
    src2 = jnp.concatenate([src, jnp.zeros((pad,), jnp.int32)]).reshape(-1, _CHUNK)
    et2 = jnp.concatenate([etypes, jnp.zeros((pad,), jnp.int32)]).reshape(-1, _CHUNK)
    dst2 = jnp.concatenate([dst, jnp.full((pad,), N, jnp.int32)]).reshape(-1, _CHUNK)

    partial = _sc_edge_aggregate(p_flat, src2, et2, dst2, R, NH, CH)
    partial = partial.reshape(NC, NH, D)

    return _tc_combine(partial[0, :N], partial[1, :N], self_term)
